# baseline (speedup 1.0000x reference)
import functools

import jax
import jax.numpy as jnp
from jax import lax
from jax.experimental import pallas as pl
from jax.experimental.pallas import tpu as pltpu
from jax.experimental.pallas import tpu_sc as plsc

BATCH = 16384
EMBED_K = 32
NUM_WORKERS = 32
ROWS_PER_WORKER = BATCH // NUM_WORKERS
CHUNK = 128
NCHUNK = ROWS_PER_WORKER // CHUNK
GROUPS = ROWS_PER_WORKER // 16


def _sc_mf_body(uidx_hbm, vidx_hbm, w_hbm, h_hbm, out_hbm,
                uidx_v, vidx_v, u_rows, v_rows, out_v, sem):
    cid = lax.axis_index("c")
    sid = lax.axis_index("s")
    wid = sid * 2 + cid
    base = wid * ROWS_PER_WORKER

    row0 = wid * NCHUNK
    pltpu.sync_copy(uidx_hbm.at[pl.ds(row0, NCHUNK)], uidx_v)
    pltpu.sync_copy(vidx_hbm.at[pl.ds(row0, NCHUNK)], vidx_v)

    copies = []
    for j in range(NCHUNK):
        copies.append(pltpu.async_copy(
            w_hbm.at[uidx_v.at[j]], u_rows.at[pl.ds(j * CHUNK, CHUNK)], sem))
        copies.append(pltpu.async_copy(
            h_hbm.at[vidx_v.at[j]], v_rows.at[pl.ds(j * CHUNK, CHUNK)], sem))
    for c in copies:
        c.wait()

    iota = lax.iota(jnp.int32, 16)

    def group_body(g, carry):
        rows = jnp.full((16,), g * 16, jnp.int32) + iota
        acc = jnp.zeros((16,), jnp.float32)
        for k in range(EMBED_K):
            colk = jnp.full((16,), k, jnp.int32)
            u = plsc.load_gather(u_rows, [rows, colk])
            v = plsc.load_gather(v_rows, [rows, colk])
            acc = acc + u * v
        out_v[pl.ds(g * 16, 16)] = acc
        return carry

    lax.fori_loop(0, GROUPS, group_body, 0)

    pltpu.sync_copy(out_v, out_hbm.at[pl.ds(base, ROWS_PER_WORKER)])


@functools.partial(
    pl.kernel,
    out_type=jax.ShapeDtypeStruct((BATCH,), jnp.float32),
    mesh=plsc.VectorSubcoreMesh(core_axis_name="c", subcore_axis_name="s"),
    compiler_params=pltpu.CompilerParams(
        needs_layout_passes=False, use_tc_tiling_on_sc=False),
    scratch_types=[
        pltpu.VMEM((NCHUNK, CHUNK), jnp.int32),
        pltpu.VMEM((NCHUNK, CHUNK), jnp.int32),
        pltpu.VMEM((ROWS_PER_WORKER, EMBED_K), jnp.float32),
        pltpu.VMEM((ROWS_PER_WORKER, EMBED_K), jnp.float32),
        pltpu.VMEM((ROWS_PER_WORKER,), jnp.float32),
        pltpu.SemaphoreType.DMA,
    ],
)
def _mf_sc(uidx_hbm, vidx_hbm, w_hbm, h_hbm, out_hbm,
           uidx_v, vidx_v, u_rows, v_rows, out_v, sem):
    _sc_mf_body(uidx_hbm, vidx_hbm, w_hbm, h_hbm, out_hbm,
                uidx_v, vidx_v, u_rows, v_rows, out_v, sem)


def kernel(x, W, H):
    uidx = x[:, 0].astype(jnp.int32).reshape(BATCH // CHUNK, CHUNK)
    vidx = x[:, 1].astype(jnp.int32).reshape(BATCH // CHUNK, CHUNK)
    return _mf_sc(uidx, vidx, W, H)
